# baseline (speedup 1.0000x reference)
import jax
import jax.numpy as jnp
from jax.experimental import pallas as pl

_TOKENS_PER_BLOCK = 2048


def _router_block(x_ref, w_ref, b_ref, tw_ref, ti_ref, acc_ref):
    x = x_ref[...]
    w = w_ref[...]
    logits = jax.lax.dot_general(
        x, w, (((1,), (1,)), ((), ())),
        preferred_element_type=jnp.float32,
        precision=jax.lax.Precision.DEFAULT,
    )
    logits = logits + b_ref[...]

    m = jnp.max(logits, axis=-1, keepdims=True)
    e = jnp.exp(logits - m)
    p = e / jnp.sum(e, axis=-1, keepdims=True)

    n_e = p.shape[-1]
    idx = jax.lax.broadcasted_iota(jnp.int32, p.shape, 1)
    w1 = jnp.max(p, axis=-1, keepdims=True)
    i1 = jnp.min(jnp.where(p == w1, idx, n_e), axis=-1, keepdims=True)
    p2 = jnp.where(idx == i1, -jnp.inf, p)
    w2 = jnp.max(p2, axis=-1, keepdims=True)
    i2 = jnp.min(jnp.where(p2 == w2, idx, n_e), axis=-1, keepdims=True)

    denom = w1 + w2 + 1e-8
    tw = jnp.concatenate([w1 / denom, w2 / denom], axis=-1)
    tw_ref[...] = tw
    ti_ref[...] = jnp.concatenate([i1, i2], axis=-1)

    s_i1 = jnp.sum(i1.astype(jnp.float32))
    s_i2 = jnp.sum(i2.astype(jnp.float32))
    s_w = jnp.sum(tw)
    lane = jax.lax.broadcasted_iota(jnp.int32, acc_ref.shape, 1)
    part = (jnp.where(lane == 0, s_i1, 0.0)
            + jnp.where(lane == 1, s_i2, 0.0)
            + jnp.where(lane == 2, s_w, 0.0))

    @pl.when(pl.program_id(0) == 0)
    def _():
        acc_ref[...] = part

    @pl.when(pl.program_id(0) != 0)
    def _():
        acc_ref[...] = acc_ref[...] + part


def kernel(hidden_states, W, b):
    B, S, D = hidden_states.shape
    T = B * S
    n_e = W.shape[0]
    x = hidden_states.reshape(T, D)
    b2 = b.reshape(1, n_e)
    tb = _TOKENS_PER_BLOCK
    tw, ti, acc = pl.pallas_call(
        _router_block,
        grid=(T // tb,),
        in_specs=[
            pl.BlockSpec((tb, D), lambda i: (i, 0)),
            pl.BlockSpec((n_e, D), lambda i: (0, 0)),
            pl.BlockSpec((1, n_e), lambda i: (0, 0)),
        ],
        out_specs=[
            pl.BlockSpec((tb, 2), lambda i: (i, 0)),
            pl.BlockSpec((tb, 2), lambda i: (i, 0)),
            pl.BlockSpec((1, 128), lambda i: (0, 0)),
        ],
        out_shape=[
            jax.ShapeDtypeStruct((T, 2), jnp.float32),
            jax.ShapeDtypeStruct((T, 2), jnp.int32),
            jax.ShapeDtypeStruct((1, 128), jnp.float32),
        ],
    )(x, W, b2)
    top_k_weights = tw.reshape(B, S, 2)
    top_k_indices = ti.reshape(B, S, 2)
    expert_usage = acc[0, :2] / T
    avg_router_confidence = acc[0, 2] / (T * 2)
    return (top_k_weights, top_k_indices, expert_usage, avg_router_confidence)


# speedup vs baseline: 1.8305x; 1.8305x over previous
import jax
import jax.numpy as jnp
from jax.experimental import pallas as pl

_TOKENS_PER_BLOCK = 2048


def _router_block(x_ref, w_ref, b_ref, tw_ref, ti_ref, acc_ref):
    x = x_ref[...]
    w = w_ref[...]
    logits = jax.lax.dot_general(
        x, w, (((1,), (1,)), ((), ())),
        preferred_element_type=jnp.float32,
        precision=jax.lax.Precision.DEFAULT,
    )
    lt = logits.T + b_ref[...].T

    m = jnp.max(lt, axis=0, keepdims=True)
    e = jnp.exp(lt - m)
    p = e / jnp.sum(e, axis=0, keepdims=True)

    n_e = p.shape[0]
    idx = jax.lax.broadcasted_iota(jnp.int32, p.shape, 0)
    w1 = jnp.max(p, axis=0, keepdims=True)
    i1 = jnp.min(jnp.where(p == w1, idx, n_e), axis=0, keepdims=True)
    p2 = jnp.where(idx == i1, -jnp.inf, p)
    w2 = jnp.max(p2, axis=0, keepdims=True)
    i2 = jnp.min(jnp.where(p2 == w2, idx, n_e), axis=0, keepdims=True)

    inv = 1.0 / (w1 + w2 + 1e-8)
    g1 = w1 * inv
    g2 = w2 * inv
    tw_ref[...] = jnp.concatenate([g1, g2], axis=0)
    ti_ref[...] = jnp.concatenate([i1, i2], axis=0)

    s_i1 = jnp.sum(i1.astype(jnp.float32))
    s_i2 = jnp.sum(i2.astype(jnp.float32))
    s_w = jnp.sum(g1) + jnp.sum(g2)
    lane = jax.lax.broadcasted_iota(jnp.int32, acc_ref.shape, 1)
    part = (jnp.where(lane == 0, s_i1, 0.0)
            + jnp.where(lane == 1, s_i2, 0.0)
            + jnp.where(lane == 2, s_w, 0.0))

    @pl.when(pl.program_id(0) == 0)
    def _():
        acc_ref[...] = part

    @pl.when(pl.program_id(0) != 0)
    def _():
        acc_ref[...] = acc_ref[...] + part


def kernel(hidden_states, W, b):
    B, S, D = hidden_states.shape
    T = B * S
    n_e = W.shape[0]
    x = hidden_states.reshape(T, D)
    b2 = b.reshape(1, n_e)
    tb = _TOKENS_PER_BLOCK
    twt, tit, acc = pl.pallas_call(
        _router_block,
        grid=(T // tb,),
        in_specs=[
            pl.BlockSpec((tb, D), lambda i: (i, 0)),
            pl.BlockSpec((n_e, D), lambda i: (0, 0)),
            pl.BlockSpec((1, n_e), lambda i: (0, 0)),
        ],
        out_specs=[
            pl.BlockSpec((2, tb), lambda i: (0, i)),
            pl.BlockSpec((2, tb), lambda i: (0, i)),
            pl.BlockSpec((1, 128), lambda i: (0, 0)),
        ],
        out_shape=[
            jax.ShapeDtypeStruct((2, T), jnp.float32),
            jax.ShapeDtypeStruct((2, T), jnp.int32),
            jax.ShapeDtypeStruct((1, 128), jnp.float32),
        ],
    )(x, W, b2)
    top_k_weights = twt.T.reshape(B, S, 2)
    top_k_indices = tit.T.reshape(B, S, 2)
    expert_usage = acc[0, :2] / T
    avg_router_confidence = acc[0, 2] / (T * 2)
    return (top_k_weights, top_k_indices, expert_usage, avg_router_confidence)
